# baseline (speedup 1.0000x reference)
import functools

import jax
import jax.numpy as jnp
from jax import lax
from jax.experimental import pallas as pl
from jax.experimental.pallas import tpu as pltpu
from jax.experimental.pallas import tpu_sc as plsc

N = 50000
E = 800000
D_IN = 1433
H1 = 64
H2 = 32

NC = 2
NS = 16
LANES = 16

CH = 128
EPAD = 802816
ECH = EPAD // CH
KB1 = 4
KB2 = 7
KD = 4
NBATCH_D = ECH // NC // NS // KD
NPAD = 50176
ROWS_PER_TILE = NPAD // NS
ZR = 196

_mesh = plsc.VectorSubcoreMesh(
    core_axis_name="c", subcore_axis_name="s", num_cores=NC, num_subcores=NS
)


def _zero_rows(ref, nrows, width):
    def body(i, _):
        for t in range(width // LANES):
            ref[i, pl.ds(t * LANES, LANES)] = jnp.zeros((LANES,), jnp.float32)
        return 0
    lax.fori_loop(0, nrows, body, 0, unroll=4)


def _deg_body(dst_hbm, out0_hbm, out1_hbm, deg_sh, idx_d2, ones_v, stage_v, sem):
    c = lax.axis_index("c")
    s = lax.axis_index("s")

    def fill_ones(i, _):
        ones_v[i, :] = jnp.ones((LANES,), jnp.float32)
        return 0
    lax.fori_loop(0, CH, fill_ones, 0, unroll=4)
    _zero_rows(stage_v, ZR, LANES)

    r0 = s * ROWS_PER_TILE

    def zero_acc(j, _):
        pltpu.sync_copy(stage_v, deg_sh.at[pl.ds(r0 + j * ZR, ZR)])
        return 0
    lax.fori_loop(0, ROWS_PER_TILE // ZR, zero_acc, 0)
    plsc.subcore_barrier()

    base = (c * NS + s) * (ECH // NC // NS)

    def edge_loop(b, _):
        r = base + b * KD
        pltpu.async_copy(dst_hbm.at[pl.ds(r, KD)], idx_d2, sem).wait()
        descs = [
            pltpu.async_copy(ones_v, deg_sh.at[idx_d2.at[j]], sem, add=True)
            for j in range(KD)
        ]
        for d in descs:
            d.wait()
        return 0
    lax.fori_loop(0, NBATCH_D, edge_loop, 0)
    plsc.subcore_barrier()

    def out_loop(j, _):
        rr = r0 + j * ZR
        pltpu.sync_copy(deg_sh.at[pl.ds(rr, ZR)], stage_v)

        @pl.when(c == 0)
        def _():
            pltpu.sync_copy(stage_v, out0_hbm.at[pl.ds(rr, ZR)])

        @pl.when(c == 1)
        def _():
            pltpu.sync_copy(stage_v, out1_hbm.at[pl.ds(rr, ZR)])
        return 0
    lax.fori_loop(0, ROWS_PER_TILE // ZR, out_loop, 0)


_deg_kernel = functools.partial(
    pl.kernel,
    out_type=[
        jax.ShapeDtypeStruct((NPAD, LANES), jnp.float32),
        jax.ShapeDtypeStruct((NPAD, LANES), jnp.float32),
    ],
    mesh=_mesh,
    compiler_params=pltpu.CompilerParams(use_tc_tiling_on_sc=False),
    scratch_types=[
        pltpu.VMEM_SHARED((NPAD, LANES), jnp.float32),
        pltpu.VMEM((KD, CH), jnp.int32),
        pltpu.VMEM((CH, LANES), jnp.float32),
        pltpu.VMEM((ZR, LANES), jnp.float32),
        pltpu.SemaphoreType.DMA,
    ],
)(_deg_body)


def _agg_body(dh, kb, mult, table_hbm, src_hbm, dst_hbm, deg0_hbm, deg1_hbm,
              outA_hbm, outB_hbm, acc_sh, idx_s2, idx_d2, rows2, d0v, d1v,
              sem_i, sem_s, *sem_g):
    c = lax.axis_index("c")
    s = lax.axis_index("s")

    _zero_rows(rows2, ZR, dh)
    r0 = s * ROWS_PER_TILE

    def zero_acc(j, _):
        pltpu.sync_copy(rows2.at[pl.ds(0, ZR)], acc_sh.at[pl.ds(r0 + j * ZR, ZR)])
        return 0
    lax.fori_loop(0, ROWS_PER_TILE // ZR, zero_acc, 0)
    plsc.subcore_barrier()

    base = s * (ECH // NS)
    nbatch = ECH // NS // kb

    pltpu.async_copy(src_hbm.at[pl.ds(base, kb)], idx_s2.at[0], sem_i)
    pltpu.async_copy(dst_hbm.at[pl.ds(base, kb)], idx_d2.at[0], sem_i)

    def do_batch(P, r, rnext):
        pltpu.make_async_copy(
            src_hbm.at[pl.ds(base, kb)], idx_s2.at[P], sem_i).wait()
        pltpu.make_async_copy(
            dst_hbm.at[pl.ds(base, kb)], idx_d2.at[P], sem_i).wait()
        for j in range(kb):
            for t in range(CH // LANES):
                sl = pl.ds(t * LANES, LANES)
                idx_s2[P, j, sl] = idx_s2[P, j, sl] * mult + c
        gd = [
            pltpu.async_copy(
                table_hbm.at[idx_s2.at[P, j]],
                rows2.at[pl.ds(j * CH, CH)], sem_g[j])
            for j in range(kb)
        ]
        pltpu.async_copy(src_hbm.at[pl.ds(rnext, kb)], idx_s2.at[1 - P], sem_i)
        pltpu.async_copy(dst_hbm.at[pl.ds(rnext, kb)], idx_d2.at[1 - P], sem_i)
        sc = []
        for j in range(kb):
            gd[j].wait()
            sc.append(pltpu.async_copy(
                rows2.at[pl.ds(j * CH, CH)],
                acc_sh.at[idx_d2.at[P, j]], sem_s, add=True))
        for d in sc:
            d.wait()

    def edge_loop(i, _):
        b0 = 2 * i
        do_batch(0, base + b0 * kb, base + (b0 + 1) * kb)
        do_batch(1, base + (b0 + 1) * kb,
                 base + ((b0 + 2) % nbatch) * kb)
        return 0
    lax.fori_loop(0, nbatch // 2, edge_loop, 0)
    pltpu.make_async_copy(
        src_hbm.at[pl.ds(base, kb)], idx_s2.at[0], sem_i).wait()
    pltpu.make_async_copy(
        dst_hbm.at[pl.ds(base, kb)], idx_d2.at[0], sem_i).wait()
    plsc.subcore_barrier()

    def out_loop(j, _):
        rr = r0 + j * ZR
        pltpu.sync_copy(acc_sh.at[pl.ds(rr, ZR)], rows2.at[pl.ds(0, ZR)])
        pltpu.sync_copy(deg0_hbm.at[pl.ds(rr, ZR)], d0v)
        pltpu.sync_copy(deg1_hbm.at[pl.ds(rr, ZR)], d1v)

        def scale_row(i, _):
            r = 1.0 / jnp.maximum(d0v[i, :] + d1v[i, :], 1.0)
            for t in range(dh // LANES):
                sl = pl.ds(t * LANES, LANES)
                rows2[i, sl] = rows2[i, sl] * r
            return 0
        lax.fori_loop(0, ZR, scale_row, 0, unroll=4)

        @pl.when(c == 0)
        def _():
            pltpu.sync_copy(rows2.at[pl.ds(0, ZR)], outA_hbm.at[pl.ds(rr, ZR)])

        @pl.when(c == 1)
        def _():
            pltpu.sync_copy(rows2.at[pl.ds(0, ZR)], outB_hbm.at[pl.ds(rr, ZR)])
        return 0
    lax.fori_loop(0, ROWS_PER_TILE // ZR, out_loop, 0)


def _make_agg(dh, kb, mult):
    return functools.partial(
        pl.kernel,
        out_type=[
            jax.ShapeDtypeStruct((NPAD, dh), jnp.float32),
            jax.ShapeDtypeStruct((NPAD, dh), jnp.float32),
        ],
        mesh=_mesh,
        compiler_params=pltpu.CompilerParams(use_tc_tiling_on_sc=False),
        scratch_types=[
            pltpu.VMEM_SHARED((NPAD, dh), jnp.float32),
            pltpu.VMEM((2, kb, CH), jnp.int32),
            pltpu.VMEM((2, kb, CH), jnp.int32),
            pltpu.VMEM((kb * CH, dh), jnp.float32),
            pltpu.VMEM((ZR, LANES), jnp.float32),
            pltpu.VMEM((ZR, LANES), jnp.float32),
            pltpu.SemaphoreType.DMA,
            pltpu.SemaphoreType.DMA,
        ] + [pltpu.SemaphoreType.DMA] * kb,
    )(functools.partial(_agg_body, dh, kb, mult))


_agg1 = _make_agg(H1 // 2, KB1, 4)
_agg2 = _make_agg(H2 // 2, KB2, 8)


BN_MM = 1024
BN_EP = 2048


def _mm1_body(xt_ref, w_ref, g1_ref):
    g1_ref[...] = lax.dot_general(
        xt_ref[...], w_ref[...], (((0,), (0,)), ((), ())),
        preferred_element_type=jnp.float32)


def _mm1(xt, w1cat):
    return pl.pallas_call(
        _mm1_body,
        grid=(pl.cdiv(N, BN_MM),),
        in_specs=[
            pl.BlockSpec((D_IN, BN_MM), lambda i: (0, i)),
            pl.BlockSpec((D_IN, 2 * H1), lambda i: (0, 0)),
        ],
        out_specs=pl.BlockSpec((BN_MM, 2 * H1), lambda i: (i, 0)),
        out_shape=jax.ShapeDtypeStruct((N, 2 * H1), jnp.float32),
    )(xt, w1cat)


def _ep1_body(accA_ref, accB_ref, g1_ref, w2_ref, b1_ref, g2_ref):
    summed = jnp.concatenate([accA_ref[...], accB_ref[...]], axis=1)
    h1 = summed + b1_ref[...] + g1_ref[...][:, H1:]
    g = jnp.dot(h1, w2_ref[...], preferred_element_type=jnp.float32)
    g2_ref[...] = jnp.concatenate(
        [g, jnp.zeros((g.shape[0], 2 * H1 - 2 * H2), jnp.float32)], axis=1)


def _ep1(accA, accB, g1, w2cat, b1r):
    return pl.pallas_call(
        _ep1_body,
        grid=(pl.cdiv(N, BN_EP),),
        in_specs=[
            pl.BlockSpec((BN_EP, H1 // 2), lambda i: (i, 0)),
            pl.BlockSpec((BN_EP, H1 // 2), lambda i: (i, 0)),
            pl.BlockSpec((BN_EP, 2 * H1), lambda i: (i, 0)),
            pl.BlockSpec((H1, H1), lambda i: (0, 0)),
            pl.BlockSpec((1, H1), lambda i: (0, 0)),
        ],
        out_specs=pl.BlockSpec((BN_EP, 2 * H1), lambda i: (i, 0)),
        out_shape=jax.ShapeDtypeStruct((N, 2 * H1), jnp.float32),
    )(accA, accB, g1, w2cat, b1r)


def _ep2_body(accA_ref, accB_ref, g2_ref, b2_ref, out_ref):
    summed = jnp.concatenate([accA_ref[...], accB_ref[...]], axis=1)
    h2 = summed + b2_ref[...] + g2_ref[...][:, H2:2 * H2]
    y = jnp.maximum(h2, 0.0)
    m = jnp.max(y, axis=1, keepdims=True)
    lse = jnp.log(jnp.sum(jnp.exp(y - m), axis=1, keepdims=True)) + m
    out_ref[...] = y - lse


def _ep2(accA, accB, g2, b2r):
    return pl.pallas_call(
        _ep2_body,
        grid=(pl.cdiv(N, BN_EP),),
        in_specs=[
            pl.BlockSpec((BN_EP, H2 // 2), lambda i: (i, 0)),
            pl.BlockSpec((BN_EP, H2 // 2), lambda i: (i, 0)),
            pl.BlockSpec((BN_EP, 2 * H1), lambda i: (i, 0)),
            pl.BlockSpec((1, H2), lambda i: (0, 0)),
        ],
        out_specs=pl.BlockSpec((BN_EP, H2), lambda i: (i, 0)),
        out_shape=jax.ShapeDtypeStruct((N, H2), jnp.float32),
    )(accA, accB, g2, b2r)


def kernel(x, edge_index, Wl1, Wr1, b1, Wl2, Wr2, b2):
    extra = EPAD - E
    src = jnp.concatenate(
        [edge_index[0], jnp.zeros((extra,), jnp.int32)]).reshape(ECH, CH)
    dst = jnp.concatenate(
        [edge_index[1], jnp.full((extra,), N, jnp.int32)]).reshape(ECH, CH)
    w1cat = jnp.concatenate([Wl1, Wr1], axis=1)
    w2cat = jnp.concatenate([Wl2, Wr2], axis=1)
    b1r = b1.reshape(1, H1)
    b2r = b2.reshape(1, H2)

    g1 = _mm1(x.T, w1cat)
    deg0, deg1 = _deg_kernel(dst)
    accA, accB = _agg1(g1.reshape(4 * N, H1 // 2), src, dst, deg0, deg1)
    g2 = _ep1(accA, accB, g1, w2cat, b1r)
    acc2A, acc2B = _agg2(g2.reshape(8 * N, H2 // 2), src, dst, deg0, deg1)
    return _ep2(acc2A, acc2B, g2, b2r)


# speedup vs baseline: 1.0082x; 1.0082x over previous
import functools

import jax
import jax.numpy as jnp
from jax import lax
from jax.experimental import pallas as pl
from jax.experimental.pallas import tpu as pltpu
from jax.experimental.pallas import tpu_sc as plsc

N = 50000
E = 800000
D_IN = 1433
H1 = 64
H2 = 32

NC = 2
NS = 16
LANES = 16

CH = 128
EPAD = 802816
ECH = EPAD // CH
KB1 = 4
KB2 = 8
KD = 4
NBATCH_D = ECH // NC // NS // KD
NPAD = 50176
ROWS_PER_TILE = NPAD // NS
ZR = 196

_mesh = plsc.VectorSubcoreMesh(
    core_axis_name="c", subcore_axis_name="s", num_cores=NC, num_subcores=NS
)


def _zero_rows(ref, nrows, width):
    def body(i, _):
        for t in range(width // LANES):
            ref[i, pl.ds(t * LANES, LANES)] = jnp.zeros((LANES,), jnp.float32)
        return 0
    lax.fori_loop(0, nrows, body, 0, unroll=4)


def _deg_body(dst_hbm, out0_hbm, out1_hbm, deg_sh, idx_d2, ones_v, stage_v, sem):
    c = lax.axis_index("c")
    s = lax.axis_index("s")

    def fill_ones(i, _):
        ones_v[i, :] = jnp.ones((LANES,), jnp.float32)
        return 0
    lax.fori_loop(0, CH, fill_ones, 0, unroll=4)
    _zero_rows(stage_v, ZR, LANES)

    r0 = s * ROWS_PER_TILE

    def zero_acc(j, _):
        pltpu.sync_copy(stage_v, deg_sh.at[pl.ds(r0 + j * ZR, ZR)])
        return 0
    lax.fori_loop(0, ROWS_PER_TILE // ZR, zero_acc, 0)
    plsc.subcore_barrier()

    base = (c * NS + s) * (ECH // NC // NS)

    def edge_loop(b, _):
        r = base + b * KD
        pltpu.async_copy(dst_hbm.at[pl.ds(r, KD)], idx_d2, sem).wait()
        descs = [
            pltpu.async_copy(ones_v, deg_sh.at[idx_d2.at[j]], sem, add=True)
            for j in range(KD)
        ]
        for d in descs:
            d.wait()
        return 0
    lax.fori_loop(0, NBATCH_D, edge_loop, 0)
    plsc.subcore_barrier()

    def out_loop(j, _):
        rr = r0 + j * ZR
        pltpu.sync_copy(deg_sh.at[pl.ds(rr, ZR)], stage_v)

        @pl.when(c == 0)
        def _():
            pltpu.sync_copy(stage_v, out0_hbm.at[pl.ds(rr, ZR)])

        @pl.when(c == 1)
        def _():
            pltpu.sync_copy(stage_v, out1_hbm.at[pl.ds(rr, ZR)])
        return 0
    lax.fori_loop(0, ROWS_PER_TILE // ZR, out_loop, 0)


_deg_kernel = functools.partial(
    pl.kernel,
    out_type=[
        jax.ShapeDtypeStruct((NPAD, LANES), jnp.float32),
        jax.ShapeDtypeStruct((NPAD, LANES), jnp.float32),
    ],
    mesh=_mesh,
    compiler_params=pltpu.CompilerParams(use_tc_tiling_on_sc=False),
    scratch_types=[
        pltpu.VMEM_SHARED((NPAD, LANES), jnp.float32),
        pltpu.VMEM((KD, CH), jnp.int32),
        pltpu.VMEM((CH, LANES), jnp.float32),
        pltpu.VMEM((ZR, LANES), jnp.float32),
        pltpu.SemaphoreType.DMA,
    ],
)(_deg_body)


def _agg_body(dh, kb, tsz, table_hbm, src_hbm, dst_hbm, deg0_hbm, deg1_hbm,
              outA_hbm, outB_hbm, acc_sh, idx_s2, idx_d2, rows2, d0v, d1v,
              sem_i, sem_s, *sem_g):
    c = lax.axis_index("c")
    s = lax.axis_index("s")
    tbl = table_hbm.at[pl.ds(c, tsz)]

    _zero_rows(rows2, ZR, dh)
    r0 = s * ROWS_PER_TILE

    def zero_acc(j, _):
        pltpu.sync_copy(rows2.at[pl.ds(0, ZR)], acc_sh.at[pl.ds(r0 + j * ZR, ZR)])
        return 0
    lax.fori_loop(0, ROWS_PER_TILE // ZR, zero_acc, 0)
    plsc.subcore_barrier()

    base = s * (ECH // NS)
    nbatch = ECH // NS // kb

    pltpu.async_copy(src_hbm.at[pl.ds(base, kb)], idx_s2.at[0], sem_i)
    pltpu.async_copy(dst_hbm.at[pl.ds(base, kb)], idx_d2.at[0], sem_i)

    def do_batch(P, r, rnext):
        pltpu.make_async_copy(
            src_hbm.at[pl.ds(base, kb)], idx_s2.at[P], sem_i).wait()
        pltpu.make_async_copy(
            dst_hbm.at[pl.ds(base, kb)], idx_d2.at[P], sem_i).wait()
        gd = [
            pltpu.async_copy(
                tbl.at[idx_s2.at[P, j]],
                rows2.at[pl.ds(j * CH, CH)], sem_g[j])
            for j in range(kb)
        ]
        pltpu.async_copy(src_hbm.at[pl.ds(rnext, kb)], idx_s2.at[1 - P], sem_i)
        pltpu.async_copy(dst_hbm.at[pl.ds(rnext, kb)], idx_d2.at[1 - P], sem_i)
        sc = []
        for j in range(kb):
            gd[j].wait()
            sc.append(pltpu.async_copy(
                rows2.at[pl.ds(j * CH, CH)],
                acc_sh.at[idx_d2.at[P, j]], sem_s, add=True))
        for d in sc:
            d.wait()

    def edge_loop(i, _):
        b0 = 2 * i
        do_batch(0, base + b0 * kb, base + (b0 + 1) * kb)
        do_batch(1, base + (b0 + 1) * kb,
                 base + ((b0 + 2) % nbatch) * kb)
        return 0
    lax.fori_loop(0, nbatch // 2, edge_loop, 0)
    if nbatch % 2:
        do_batch(0, base + (nbatch - 1) * kb, base)
    pf = nbatch % 2
    pltpu.make_async_copy(
        src_hbm.at[pl.ds(base, kb)], idx_s2.at[pf], sem_i).wait()
    pltpu.make_async_copy(
        dst_hbm.at[pl.ds(base, kb)], idx_d2.at[pf], sem_i).wait()
    plsc.subcore_barrier()

    def out_loop(j, _):
        rr = r0 + j * ZR
        pltpu.sync_copy(acc_sh.at[pl.ds(rr, ZR)], rows2.at[pl.ds(0, ZR)])
        pltpu.sync_copy(deg0_hbm.at[pl.ds(rr, ZR)], d0v)
        pltpu.sync_copy(deg1_hbm.at[pl.ds(rr, ZR)], d1v)

        def scale_row(i, _):
            r = 1.0 / jnp.maximum(d0v[i, :] + d1v[i, :], 1.0)
            for t in range(dh // LANES):
                sl = pl.ds(t * LANES, LANES)
                rows2[i, sl] = rows2[i, sl] * r
            return 0
        lax.fori_loop(0, ZR, scale_row, 0, unroll=4)

        @pl.when(c == 0)
        def _():
            pltpu.sync_copy(rows2.at[pl.ds(0, ZR)], outA_hbm.at[pl.ds(rr, ZR)])

        @pl.when(c == 1)
        def _():
            pltpu.sync_copy(rows2.at[pl.ds(0, ZR)], outB_hbm.at[pl.ds(rr, ZR)])
        return 0
    lax.fori_loop(0, ROWS_PER_TILE // ZR, out_loop, 0)


def _make_agg(dh, kb, tsz):
    return functools.partial(
        pl.kernel,
        out_type=[
            jax.ShapeDtypeStruct((NPAD, dh), jnp.float32),
            jax.ShapeDtypeStruct((NPAD, dh), jnp.float32),
        ],
        mesh=_mesh,
        compiler_params=pltpu.CompilerParams(use_tc_tiling_on_sc=False),
        scratch_types=[
            pltpu.VMEM_SHARED((NPAD, dh), jnp.float32),
            pltpu.VMEM((2, kb, CH), jnp.int32),
            pltpu.VMEM((2, kb, CH), jnp.int32),
            pltpu.VMEM((kb * CH, dh), jnp.float32),
            pltpu.VMEM((ZR, LANES), jnp.float32),
            pltpu.VMEM((ZR, LANES), jnp.float32),
            pltpu.SemaphoreType.DMA,
            pltpu.SemaphoreType.DMA,
        ] + [pltpu.SemaphoreType.DMA] * kb,
    )(functools.partial(_agg_body, dh, kb, tsz))


_agg1 = _make_agg(H1 // 2, KB1, 4 * NPAD - 8)
_agg2 = _make_agg(H2 // 2, KB2, 8 * NPAD - 8)


BN_MM = 1024
BN_EP = 2048


def _mm1_body(xt_ref, w_ref, g1_ref):
    g1_ref[...] = lax.dot_general(
        xt_ref[...], w_ref[...], (((0,), (0,)), ((), ())),
        preferred_element_type=jnp.float32)


def _mm1(xt, w1cat):
    return pl.pallas_call(
        _mm1_body,
        grid=(pl.cdiv(N, BN_MM),),
        in_specs=[
            pl.BlockSpec((D_IN, BN_MM), lambda i: (0, i)),
            pl.BlockSpec((D_IN, 2 * H1), lambda i: (0, 0)),
        ],
        out_specs=pl.BlockSpec((BN_MM, 2 * H1), lambda i: (i, 0)),
        out_shape=jax.ShapeDtypeStruct((NPAD, 2 * H1), jnp.float32),
    )(xt, w1cat)


def _ep1_body(accA_ref, accB_ref, g1_ref, w2_ref, b1_ref, g2_ref):
    summed = jnp.concatenate([accA_ref[...], accB_ref[...]], axis=1)
    h1 = summed + b1_ref[...] + g1_ref[...][:, H1:]
    g = jnp.dot(h1, w2_ref[...], preferred_element_type=jnp.float32)
    g2_ref[...] = jnp.concatenate(
        [g, jnp.zeros((g.shape[0], 2 * H1 - 2 * H2), jnp.float32)], axis=1)


def _ep1(accA, accB, g1, w2cat, b1r):
    return pl.pallas_call(
        _ep1_body,
        grid=(pl.cdiv(N, BN_EP),),
        in_specs=[
            pl.BlockSpec((BN_EP, H1 // 2), lambda i: (i, 0)),
            pl.BlockSpec((BN_EP, H1 // 2), lambda i: (i, 0)),
            pl.BlockSpec((BN_EP, 2 * H1), lambda i: (i, 0)),
            pl.BlockSpec((H1, H1), lambda i: (0, 0)),
            pl.BlockSpec((1, H1), lambda i: (0, 0)),
        ],
        out_specs=pl.BlockSpec((BN_EP, 2 * H1), lambda i: (i, 0)),
        out_shape=jax.ShapeDtypeStruct((NPAD, 2 * H1), jnp.float32),
    )(accA, accB, g1, w2cat, b1r)


def _ep2_body(accA_ref, accB_ref, g2_ref, b2_ref, out_ref):
    summed = jnp.concatenate([accA_ref[...], accB_ref[...]], axis=1)
    h2 = summed + b2_ref[...] + g2_ref[...][:, H2:2 * H2]
    y = jnp.maximum(h2, 0.0)
    m = jnp.max(y, axis=1, keepdims=True)
    lse = jnp.log(jnp.sum(jnp.exp(y - m), axis=1, keepdims=True)) + m
    out_ref[...] = y - lse


def _ep2(accA, accB, g2, b2r):
    return pl.pallas_call(
        _ep2_body,
        grid=(pl.cdiv(N, BN_EP),),
        in_specs=[
            pl.BlockSpec((BN_EP, H2 // 2), lambda i: (i, 0)),
            pl.BlockSpec((BN_EP, H2 // 2), lambda i: (i, 0)),
            pl.BlockSpec((BN_EP, 2 * H1), lambda i: (i, 0)),
            pl.BlockSpec((1, H2), lambda i: (0, 0)),
        ],
        out_specs=pl.BlockSpec((BN_EP, H2), lambda i: (i, 0)),
        out_shape=jax.ShapeDtypeStruct((N, H2), jnp.float32),
    )(accA, accB, g2, b2r)


def kernel(x, edge_index, Wl1, Wr1, b1, Wl2, Wr2, b2):
    extra = EPAD - E
    src = jnp.concatenate(
        [edge_index[0], jnp.zeros((extra,), jnp.int32)]).reshape(ECH, CH)
    dst = jnp.concatenate(
        [edge_index[1], jnp.full((extra,), N, jnp.int32)]).reshape(ECH, CH)
    w1cat = jnp.concatenate([Wl1, Wr1], axis=1)
    w2cat = jnp.concatenate([Wl2, Wr2], axis=1)
    b1r = b1.reshape(1, H1)
    b2r = b2.reshape(1, H2)

    src4 = src * 4
    src8 = src * 8

    g1 = _mm1(x.T, w1cat)
    deg0, deg1 = _deg_kernel(dst)
    accA, accB = _agg1(g1.reshape(4 * NPAD, H1 // 2), src4, dst, deg0, deg1)
    g2 = _ep1(accA, accB, g1, w2cat, b1r)
    acc2A, acc2B = _agg2(g2.reshape(8 * NPAD, H2 // 2), src8, dst, deg0, deg1)
    return _ep2(acc2A, acc2B, g2, b2r)


# speedup vs baseline: 1.0413x; 1.0329x over previous
import functools

import jax
import jax.numpy as jnp
from jax import lax
from jax.experimental import pallas as pl
from jax.experimental.pallas import tpu as pltpu
from jax.experimental.pallas import tpu_sc as plsc

N = 50000
E = 800000
D_IN = 1433
H1 = 64
H2 = 32

NC = 2
NS = 16
LANES = 16

CH = 128
EPAD = 802816
ECH = EPAD // CH
KB1 = 4
KB2 = 8
KD = 4
NBATCH_D = ECH // NC // NS // KD
NPAD = 50176
ROWS_PER_TILE = NPAD // NS
ZR = 196

_mesh = plsc.VectorSubcoreMesh(
    core_axis_name="c", subcore_axis_name="s", num_cores=NC, num_subcores=NS
)


def _zero_rows(ref, nrows, width):
    def body(i, _):
        for t in range(width // LANES):
            ref[i, pl.ds(t * LANES, LANES)] = jnp.zeros((LANES,), jnp.float32)
        return 0
    lax.fori_loop(0, nrows, body, 0, unroll=4)


def _deg_body(dst_hbm, out0_hbm, out1_hbm, deg_sh, idx_d2, ones_v, stage_v, sem):
    c = lax.axis_index("c")
    s = lax.axis_index("s")

    def fill_ones(i, _):
        ones_v[i, :] = jnp.ones((LANES,), jnp.float32)
        return 0
    lax.fori_loop(0, CH, fill_ones, 0, unroll=4)
    _zero_rows(stage_v, ZR, LANES)

    r0 = s * ROWS_PER_TILE

    def zero_acc(j, _):
        pltpu.sync_copy(stage_v, deg_sh.at[pl.ds(r0 + j * ZR, ZR)])
        return 0
    lax.fori_loop(0, ROWS_PER_TILE // ZR, zero_acc, 0)
    plsc.subcore_barrier()

    base = (c * NS + s) * (ECH // NC // NS)

    def edge_loop(b, _):
        r = base + b * KD
        pltpu.async_copy(dst_hbm.at[pl.ds(r, KD)], idx_d2, sem).wait()
        descs = [
            pltpu.async_copy(ones_v, deg_sh.at[idx_d2.at[j]], sem, add=True)
            for j in range(KD)
        ]
        for d in descs:
            d.wait()
        return 0
    lax.fori_loop(0, NBATCH_D, edge_loop, 0)
    plsc.subcore_barrier()

    def out_loop(j, _):
        rr = r0 + j * ZR
        pltpu.sync_copy(deg_sh.at[pl.ds(rr, ZR)], stage_v)

        @pl.when(c == 0)
        def _():
            pltpu.sync_copy(stage_v, out0_hbm.at[pl.ds(rr, ZR)])

        @pl.when(c == 1)
        def _():
            pltpu.sync_copy(stage_v, out1_hbm.at[pl.ds(rr, ZR)])
        return 0
    lax.fori_loop(0, ROWS_PER_TILE // ZR, out_loop, 0)


_deg_kernel = functools.partial(
    pl.kernel,
    out_type=[
        jax.ShapeDtypeStruct((NPAD, LANES), jnp.float32),
        jax.ShapeDtypeStruct((NPAD, LANES), jnp.float32),
    ],
    mesh=_mesh,
    compiler_params=pltpu.CompilerParams(use_tc_tiling_on_sc=False),
    scratch_types=[
        pltpu.VMEM_SHARED((NPAD, LANES), jnp.float32),
        pltpu.VMEM((KD, CH), jnp.int32),
        pltpu.VMEM((CH, LANES), jnp.float32),
        pltpu.VMEM((ZR, LANES), jnp.float32),
        pltpu.SemaphoreType.DMA,
    ],
)(_deg_body)


def _agg_body(dh, kb, tsz, table_hbm, src_hbm, dst_hbm, deg0_hbm, deg1_hbm,
              outA_hbm, outB_hbm, acc_sh, idx_s2, idx_d2, rows2, d0v, d1v,
              sem_i, sem_s, *sem_g):
    c = lax.axis_index("c")
    s = lax.axis_index("s")
    tbl = table_hbm.at[pl.ds(c, tsz)]

    _zero_rows(rows2, ZR, dh)
    r0 = s * ROWS_PER_TILE

    def zero_acc(j, _):
        pltpu.sync_copy(rows2.at[pl.ds(0, ZR)], acc_sh.at[pl.ds(r0 + j * ZR, ZR)])
        return 0
    lax.fori_loop(0, ROWS_PER_TILE // ZR, zero_acc, 0)
    plsc.subcore_barrier()

    base = s * (ECH // NS)
    nbatch = ECH // NS // kb

    pltpu.async_copy(src_hbm.at[pl.ds(base, kb)], idx_s2.at[0], sem_i)
    pltpu.async_copy(dst_hbm.at[pl.ds(base, kb)], idx_d2.at[0], sem_i)

    def do_batch(P, r, rnext):
        pltpu.make_async_copy(
            src_hbm.at[pl.ds(base, kb)], idx_s2.at[P], sem_i).wait()
        pltpu.make_async_copy(
            dst_hbm.at[pl.ds(base, kb)], idx_d2.at[P], sem_i).wait()
        gd = [
            pltpu.async_copy(
                tbl.at[idx_s2.at[P, j]],
                rows2.at[pl.ds(j * CH, CH)], sem_g[j])
            for j in range(kb)
        ]
        pltpu.async_copy(src_hbm.at[pl.ds(rnext, kb)], idx_s2.at[1 - P], sem_i)
        pltpu.async_copy(dst_hbm.at[pl.ds(rnext, kb)], idx_d2.at[1 - P], sem_i)
        sc = []
        for j in range(kb):
            gd[j].wait()
            sc.append(pltpu.async_copy(
                rows2.at[pl.ds(j * CH, CH)],
                acc_sh.at[idx_d2.at[P, j]], sem_s, add=True))
        for d in sc:
            d.wait()

    def edge_loop(i, _):
        b0 = 2 * i
        do_batch(0, base + b0 * kb, base + (b0 + 1) * kb)
        do_batch(1, base + (b0 + 1) * kb,
                 base + ((b0 + 2) % nbatch) * kb)
        return 0
    lax.fori_loop(0, nbatch // 2, edge_loop, 0)
    if nbatch % 2:
        do_batch(0, base + (nbatch - 1) * kb, base)
    pf = nbatch % 2
    pltpu.make_async_copy(
        src_hbm.at[pl.ds(base, kb)], idx_s2.at[pf], sem_i).wait()
    pltpu.make_async_copy(
        dst_hbm.at[pl.ds(base, kb)], idx_d2.at[pf], sem_i).wait()
    plsc.subcore_barrier()

    def out_loop(j, _):
        rr = r0 + j * ZR
        da = pltpu.async_copy(acc_sh.at[pl.ds(rr, ZR)], rows2.at[pl.ds(0, ZR)], sem_g[0])
        d0 = pltpu.async_copy(deg0_hbm.at[pl.ds(rr, ZR)], d0v, sem_g[1])
        d1 = pltpu.async_copy(deg1_hbm.at[pl.ds(rr, ZR)], d1v, sem_g[2])
        da.wait()
        d0.wait()
        d1.wait()

        def scale_row(i, _):
            r = 1.0 / jnp.maximum(d0v[i, :] + d1v[i, :], 1.0)
            for t in range(dh // LANES):
                sl = pl.ds(t * LANES, LANES)
                rows2[i, sl] = rows2[i, sl] * r
            return 0
        lax.fori_loop(0, ZR, scale_row, 0, unroll=4)

        @pl.when(c == 0)
        def _():
            pltpu.sync_copy(rows2.at[pl.ds(0, ZR)], outA_hbm.at[pl.ds(rr, ZR)])

        @pl.when(c == 1)
        def _():
            pltpu.sync_copy(rows2.at[pl.ds(0, ZR)], outB_hbm.at[pl.ds(rr, ZR)])
        return 0
    lax.fori_loop(0, ROWS_PER_TILE // ZR, out_loop, 0)


def _make_agg(dh, kb, tsz):
    return functools.partial(
        pl.kernel,
        out_type=[
            jax.ShapeDtypeStruct((NPAD, dh), jnp.float32),
            jax.ShapeDtypeStruct((NPAD, dh), jnp.float32),
        ],
        mesh=_mesh,
        compiler_params=pltpu.CompilerParams(use_tc_tiling_on_sc=False),
        scratch_types=[
            pltpu.VMEM_SHARED((NPAD, dh), jnp.float32),
            pltpu.VMEM((2, kb, CH), jnp.int32),
            pltpu.VMEM((2, kb, CH), jnp.int32),
            pltpu.VMEM((kb * CH, dh), jnp.float32),
            pltpu.VMEM((ZR, LANES), jnp.float32),
            pltpu.VMEM((ZR, LANES), jnp.float32),
            pltpu.SemaphoreType.DMA,
            pltpu.SemaphoreType.DMA,
        ] + [pltpu.SemaphoreType.DMA] * kb,
    )(functools.partial(_agg_body, dh, kb, tsz))


_agg1 = _make_agg(H1 // 2, KB1, 4 * NPAD - 8)
_agg2 = _make_agg(H2 // 2, KB2, 8 * NPAD - 8)


BN_MM = 1024
BN_EP = 2048


def _mm1_body(xt_ref, w_ref, g1_ref):
    g1_ref[...] = lax.dot_general(
        xt_ref[...], w_ref[...], (((0,), (0,)), ((), ())),
        preferred_element_type=jnp.float32)


def _mm1(xt, w1cat):
    return pl.pallas_call(
        _mm1_body,
        grid=(pl.cdiv(N, BN_MM),),
        in_specs=[
            pl.BlockSpec((D_IN, BN_MM), lambda i: (0, i)),
            pl.BlockSpec((D_IN, 2 * H1), lambda i: (0, 0)),
        ],
        out_specs=pl.BlockSpec((BN_MM, 2 * H1), lambda i: (i, 0)),
        out_shape=jax.ShapeDtypeStruct((NPAD, 2 * H1), jnp.float32),
    )(xt, w1cat)


def _ep1_body(accA_ref, accB_ref, g1_ref, w2_ref, b1_ref, g2_ref):
    summed = jnp.concatenate([accA_ref[...], accB_ref[...]], axis=1)
    h1 = summed + b1_ref[...] + g1_ref[...][:, H1:]
    g = jnp.dot(h1, w2_ref[...], preferred_element_type=jnp.float32)
    g2_ref[...] = jnp.concatenate(
        [g, jnp.zeros((g.shape[0], 2 * H1 - 2 * H2), jnp.float32)], axis=1)


def _ep1(accA, accB, g1, w2cat, b1r):
    return pl.pallas_call(
        _ep1_body,
        grid=(pl.cdiv(N, BN_EP),),
        in_specs=[
            pl.BlockSpec((BN_EP, H1 // 2), lambda i: (i, 0)),
            pl.BlockSpec((BN_EP, H1 // 2), lambda i: (i, 0)),
            pl.BlockSpec((BN_EP, 2 * H1), lambda i: (i, 0)),
            pl.BlockSpec((H1, H1), lambda i: (0, 0)),
            pl.BlockSpec((1, H1), lambda i: (0, 0)),
        ],
        out_specs=pl.BlockSpec((BN_EP, 2 * H1), lambda i: (i, 0)),
        out_shape=jax.ShapeDtypeStruct((NPAD, 2 * H1), jnp.float32),
    )(accA, accB, g1, w2cat, b1r)


def _ep2_body(accA_ref, accB_ref, g2_ref, b2_ref, out_ref):
    summed = jnp.concatenate([accA_ref[...], accB_ref[...]], axis=1)
    h2 = summed + b2_ref[...] + g2_ref[...][:, H2:2 * H2]
    y = jnp.maximum(h2, 0.0)
    m = jnp.max(y, axis=1, keepdims=True)
    lse = jnp.log(jnp.sum(jnp.exp(y - m), axis=1, keepdims=True)) + m
    out_ref[...] = y - lse


def _ep2(accA, accB, g2, b2r):
    return pl.pallas_call(
        _ep2_body,
        grid=(pl.cdiv(N, BN_EP),),
        in_specs=[
            pl.BlockSpec((BN_EP, H2 // 2), lambda i: (i, 0)),
            pl.BlockSpec((BN_EP, H2 // 2), lambda i: (i, 0)),
            pl.BlockSpec((BN_EP, 2 * H1), lambda i: (i, 0)),
            pl.BlockSpec((1, H2), lambda i: (0, 0)),
        ],
        out_specs=pl.BlockSpec((BN_EP, H2), lambda i: (i, 0)),
        out_shape=jax.ShapeDtypeStruct((N, H2), jnp.float32),
    )(accA, accB, g2, b2r)


def kernel(x, edge_index, Wl1, Wr1, b1, Wl2, Wr2, b2):
    extra = EPAD - E
    src = jnp.concatenate(
        [edge_index[0], jnp.zeros((extra,), jnp.int32)]).reshape(ECH, CH)
    dst = jnp.concatenate(
        [edge_index[1], jnp.full((extra,), N, jnp.int32)]).reshape(ECH, CH)
    w1cat = jnp.concatenate([Wl1, Wr1], axis=1)
    w2cat = jnp.concatenate([Wl2, Wr2], axis=1)
    b1r = b1.reshape(1, H1)
    b2r = b2.reshape(1, H2)

    src4 = src * 4
    src8 = src * 8

    g1 = _mm1(x.T, w1cat)
    deg0, deg1 = _deg_kernel(dst)
    accA, accB = _agg1(g1.reshape(4 * NPAD, H1 // 2), src4, dst, deg0, deg1)
    g2 = _ep1(accA, accB, g1, w2cat, b1r)
    acc2A, acc2B = _agg2(g2.reshape(8 * NPAD, H2 // 2), src8, dst, deg0, deg1)
    return _ep2(acc2A, acc2B, g2, b2r)
